# Initial kernel scaffold; baseline (speedup 1.0000x reference)
import jax
import jax.numpy as jnp
from jax.experimental import pallas as pl


def kernel(x, edge_index, edge_attr, params):
    raise NotImplementedError("write your pallas kernel here")


# speedup vs baseline: 2.7582x; 2.7582x over previous
import functools

import jax
import jax.numpy as jnp
from jax import lax
from jax.experimental import pallas as pl
from jax.experimental.pallas import tpu as pltpu
from jax.experimental.pallas import tpu_sc as plsc

NN = 10000
EE = 320000
HH = 128

NC = 2
NS = 16
NW = NC * NS
PER_W = EE // NW
CH = 80
NPAD = 10240
N_PER_S = NPAD // NS

EBLK = 2000
NBLK = 2000

_F32 = jnp.float32


def _dot(a, w):
    return jnp.dot(a, w, preferred_element_type=_F32)


def _ln(z, g, b):
    m = jnp.mean(z, axis=-1, keepdims=True)
    v = jnp.mean((z - m) * (z - m), axis=-1, keepdims=True)
    return (z - m) * lax.rsqrt(v + 1e-5) * g + b


def _full_spec(shape):
    nd = len(shape)
    return pl.BlockSpec(shape, lambda i, *, _nd=nd: (0,) * _nd)


def _row_spec(blk, width):
    return pl.BlockSpec((blk, width), lambda i: (i, 0))


def _edge_enc_body(ea, w1, b1, w2, b2, w3, b3, g, bn, out):
    z = jnp.maximum(_dot(ea[...], w1[...]) + b1[...], 0.0)
    z = jnp.maximum(_dot(z, w2[...]) + b2[...], 0.0)
    z = _dot(z, w3[...]) + b3[...]
    out[...] = _ln(z, g[...], bn[...])


def _node_enc_body(x, w1, b1, w2, b2, w3, b3, g, bn, wa, ba, wb, h_out, a_out, b_out):
    z = jnp.maximum(_dot(x[...], w1[...]) + b1[...], 0.0)
    z = jnp.maximum(_dot(z, w2[...]) + b2[...], 0.0)
    z = _dot(z, w3[...]) + b3[...]
    h = _ln(z, g[...], bn[...])
    h_out[...] = h
    a_out[...] = _dot(h, wa[...]) + ba[...]
    b_out[...] = _dot(h, wb[...])


def _edge_round_body(ga, gb, e, w1c, w2, b2, w3, b3, g, bn, out):
    ev = e[...]
    z = jnp.maximum(ga[...] + gb[...] + _dot(ev, w1c[...]), 0.0)
    z = jnp.maximum(_dot(z, w2[...]) + b2[...], 0.0)
    z = _dot(z, w3[...]) + b3[...]
    out[...] = _ln(z, g[...], bn[...]) + ev


def _node_round_body(h, parts, v1a, v1b, c1, v2, c2, v3, c3, g, bn,
                     wa, ba, wb, h_out, a_out, b_out):
    hv = h[...]
    agg = parts[0] + parts[1]
    z = jnp.maximum(_dot(hv, v1a[...]) + _dot(agg, v1b[...]) + c1[...], 0.0)
    z = jnp.maximum(_dot(z, v2[...]) + c2[...], 0.0)
    z = _dot(z, v3[...]) + c3[...]
    hn = _ln(z, g[...], bn[...]) + hv
    h_out[...] = hn
    a_out[...] = _dot(hn, wa[...]) + ba[...]
    b_out[...] = _dot(hn, wb[...])


def _node_last_body(h, parts, v1a, v1b, c1, v2, c2, v3, c3, g, bn, h_out):
    hv = h[...]
    agg = parts[0] + parts[1]
    z = jnp.maximum(_dot(hv, v1a[...]) + _dot(agg, v1b[...]) + c1[...], 0.0)
    z = jnp.maximum(_dot(z, v2[...]) + c2[...], 0.0)
    z = _dot(z, v3[...]) + c3[...]
    h_out[...] = _ln(z, g[...], bn[...]) + hv


def _decoder_body(h, w1, b1, w2, b2, w3, b3, out):
    z = jnp.maximum(_dot(h[...], w1[...]) + b1[...], 0.0)
    z = jnp.maximum(_dot(z, w2[...]) + b2[...], 0.0)
    out[...] = _dot(z, w3[...]) + b3[...]


def _wspecs(n):
    return [_full_spec((HH, HH)) if s == "w" else _full_spec((1, HH)) for s in n]


def _call_edge_enc(ea, p):
    grid = (EE // EBLK,)
    return pl.pallas_call(
        _edge_enc_body,
        grid=grid,
        in_specs=[_row_spec(EBLK, 16), _full_spec((16, HH))] + _wspecs("bwbwbbb"),
        out_specs=_row_spec(EBLK, HH),
        out_shape=jax.ShapeDtypeStruct((EE, HH), _F32),
    )(ea, *p)


def _call_node_enc(x, p):
    grid = (NN // NBLK,)
    spec = _row_spec(NBLK, HH)
    return pl.pallas_call(
        _node_enc_body,
        grid=grid,
        in_specs=[spec] + _wspecs("wbwbwbbb") + _wspecs("wbw"),
        out_specs=[spec, spec, spec],
        out_shape=[jax.ShapeDtypeStruct((NN, HH), _F32)] * 3,
    )(x, *p)


def _call_edge_round(ga, gb, e, p):
    grid = (EE // EBLK,)
    spec = _row_spec(EBLK, HH)
    return pl.pallas_call(
        _edge_round_body,
        grid=grid,
        in_specs=[spec, spec, spec] + _wspecs("wwbwbbb"),
        out_specs=spec,
        out_shape=jax.ShapeDtypeStruct((EE, HH), _F32),
    )(ga, gb, e, *p)


def _call_node_round(h, parts, p, last):
    grid = (NN // NBLK,)
    spec = _row_spec(NBLK, HH)
    pspec = pl.BlockSpec((2, NBLK, HH), lambda i: (0, i, 0))
    if last:
        return pl.pallas_call(
            _node_last_body,
            grid=grid,
            in_specs=[spec, pspec] + _wspecs("wwbwbwbbb"),
            out_specs=spec,
            out_shape=jax.ShapeDtypeStruct((NN, HH), _F32),
        )(h, parts, *p)
    return pl.pallas_call(
        _node_round_body,
        grid=grid,
        in_specs=[spec, pspec] + _wspecs("wwbwbwbbb") + _wspecs("wbw"),
        out_specs=[spec, spec, spec],
        out_shape=[jax.ShapeDtypeStruct((NN, HH), _F32)] * 3,
    )(h, parts, *p)


def _call_decoder(h, p):
    grid = (NN // NBLK,)
    spec = _row_spec(NBLK, HH)
    return pl.pallas_call(
        _decoder_body,
        grid=grid,
        in_specs=[spec] + _wspecs("wbwbwb"),
        out_specs=spec,
        out_shape=jax.ShapeDtypeStruct((NN, HH), _F32),
    )(h, *p)


def _sc_mesh():
    return plsc.VectorSubcoreMesh(
        core_axis_name="c", subcore_axis_name="s", num_cores=NC, num_subcores=NS)


def _sc_gather_body(hs_a, hs_b, src, dst, ga_out, gb_out,
                    srcv, dstv, bufa, bufb, sema, semb):
    wid = lax.axis_index("s") * NC + lax.axis_index("c")
    base = wid * PER_W

    def body(j, carry):
        off = base + j * CH
        pltpu.sync_copy(src.at[pl.ds(off, CH)], srcv)
        pltpu.sync_copy(dst.at[pl.ds(off, CH)], dstv)
        ca = pltpu.async_copy(hs_a.at[srcv], bufa, sema)
        cb = pltpu.async_copy(hs_b.at[dstv], bufb, semb)
        ca.wait()
        cb.wait()
        pltpu.sync_copy(bufa, ga_out.at[pl.ds(off, CH)])
        pltpu.sync_copy(bufb, gb_out.at[pl.ds(off, CH)])
        return carry

    lax.fori_loop(0, PER_W // CH, body, 0)


def _sc_gather(hs_a, hs_b, src, dst):
    k = pl.kernel(
        _sc_gather_body,
        out_type=[
            jax.ShapeDtypeStruct((EE, HH), _F32),
            jax.ShapeDtypeStruct((EE, HH), _F32),
        ],
        mesh=_sc_mesh(),
        scratch_types=[
            pltpu.VMEM((CH,), jnp.int32),
            pltpu.VMEM((CH,), jnp.int32),
            pltpu.VMEM((CH, HH), _F32),
            pltpu.VMEM((CH, HH), _F32),
            pltpu.SemaphoreType.DMA,
            pltpu.SemaphoreType.DMA,
        ],
    )
    return k(hs_a, hs_b, src, dst)


def _sc_scatter_body(e_new, dst, zeros, out, dstv, rows, agg_sh, sem):
    cid = lax.axis_index("c")
    sid = lax.axis_index("s")
    wid = sid * NC + cid
    base = wid * PER_W

    pltpu.sync_copy(zeros, agg_sh.at[pl.ds(sid * N_PER_S, N_PER_S)])
    plsc.subcore_barrier()

    def body(j, carry):
        off = base + j * CH
        pltpu.sync_copy(dst.at[pl.ds(off, CH)], dstv)
        pltpu.async_copy(e_new.at[pl.ds(off, CH)], rows, sem).wait()
        pltpu.sync_copy(rows, agg_sh.at[dstv], add=True)
        return carry

    lax.fori_loop(0, PER_W // CH, body, 0)

    plsc.subcore_barrier()
    pltpu.sync_copy(
        agg_sh.at[pl.ds(sid * N_PER_S, N_PER_S)],
        out.at[cid, pl.ds(sid * N_PER_S, N_PER_S)],
    )


def _sc_scatter(e_new, dst, zeros):
    k = pl.kernel(
        _sc_scatter_body,
        out_type=jax.ShapeDtypeStruct((NC, NPAD, HH), _F32),
        mesh=_sc_mesh(),
        scratch_types=[
            pltpu.VMEM((CH,), jnp.int32),
            pltpu.VMEM((CH, HH), _F32),
            pltpu.VMEM_SHARED((NPAD, HH), _F32),
            pltpu.SemaphoreType.DMA,
        ],
    )
    return k(e_new, dst, zeros)


def _mlp_params(p, ln):
    ls = p["layers"]
    out = []
    for l in ls:
        out.append(l["W"])
        out.append(l["b"].reshape(1, -1))
    if ln:
        out.append(p["ln"]["g"].reshape(1, -1))
        out.append(p["ln"]["b"].reshape(1, -1))
    return out


def kernel(x, edge_index, edge_attr, params):
    src = edge_index[0]
    dst = edge_index[1]

    enc_e = _mlp_params(params["edge_enc"], True)
    enc_n = _mlp_params(params["node_enc"], True)
    dec = _mlp_params(params["decoder"], False)
    w3d = jnp.zeros((HH, HH), _F32).at[:, :3].set(dec[4])
    b3d = jnp.zeros((1, HH), _F32).at[:, :3].set(dec[5])
    dec = dec[:4] + [w3d, b3d]

    blocks = []
    for bp in params["blocks"]:
        em = _mlp_params(bp["edge_mlp"], True)
        w1 = em[0]
        blk = {
            "wa": w1[:HH],
            "ba": em[1],
            "wb": w1[HH:2 * HH],
            "edge": [w1[2 * HH:]] + em[2:],
        }
        nm = _mlp_params(bp["node_mlp"], True)
        v1 = nm[0]
        blk["node"] = [v1[:HH], v1[HH:]] + nm[1:]
        blocks.append(blk)

    e = _call_edge_enc(edge_attr, enc_e)
    b0 = blocks[0]
    h, hs_a, hs_b = _call_node_enc(x, enc_n + [b0["wa"], b0["ba"], b0["wb"]])

    zeros = jnp.zeros((N_PER_S, HH), _F32)

    for r in range(15):
        blk = blocks[r]
        ga, gb = _sc_gather(hs_a, hs_b, src, dst)
        e = _call_edge_round(ga, gb, e, blk["edge"])
        parts = _sc_scatter(e, dst, zeros)[:, :NN]
        if r + 1 < 15:
            nxt = blocks[r + 1]
            h, hs_a, hs_b = _call_node_round(
                h, parts, blk["node"] + [nxt["wa"], nxt["ba"], nxt["wb"]], False)
        else:
            h = _call_node_round(h, parts, blk["node"], True)

    out = _call_decoder(h, dec)
    return out[:, :3]


# speedup vs baseline: 4.1527x; 1.5056x over previous
import functools

import jax
import jax.numpy as jnp
from jax import lax
from jax.experimental import pallas as pl
from jax.experimental.pallas import tpu as pltpu
from jax.experimental.pallas import tpu_sc as plsc

NN = 10000
EE = 320000
HH = 128

NC = 2
NS = 16
NW = NC * NS
PER_W = EE // NW
CH = 80
NPAD = 10240
N_PER_S = NPAD // NS

EBLK = 2000
NBLK = 2000

_F32 = jnp.float32


def _dot(a, w):
    return jnp.dot(a, w, preferred_element_type=_F32)


def _ln(z, g, b):
    m = jnp.mean(z, axis=-1, keepdims=True)
    v = jnp.mean((z - m) * (z - m), axis=-1, keepdims=True)
    return (z - m) * lax.rsqrt(v + 1e-5) * g + b


def _full_spec(shape):
    nd = len(shape)
    return pl.BlockSpec(shape, lambda i, *, _nd=nd: (0,) * _nd)


def _row_spec(blk, width):
    return pl.BlockSpec((blk, width), lambda i: (i, 0))


def _edge_enc_body(ea, w1, b1, w2, b2, w3, b3, g, bn, out):
    z = jnp.maximum(_dot(ea[...], w1[...]) + b1[...], 0.0)
    z = jnp.maximum(_dot(z, w2[...]) + b2[...], 0.0)
    z = _dot(z, w3[...]) + b3[...]
    out[...] = _ln(z, g[...], bn[...])


def _node_enc_body(x, w1, b1, w2, b2, w3, b3, g, bn, wa, ba, wb, h_out, a_out, b_out):
    z = jnp.maximum(_dot(x[...], w1[...]) + b1[...], 0.0)
    z = jnp.maximum(_dot(z, w2[...]) + b2[...], 0.0)
    z = _dot(z, w3[...]) + b3[...]
    h = _ln(z, g[...], bn[...])
    h_out[...] = h
    a_out[...] = _dot(h, wa[...]) + ba[...]
    b_out[...] = _dot(h, wb[...])


def _edge_round_body(ga, gb, e, w1c, w2, b2, w3, b3, g, bn, out):
    ev = e[...]
    z = jnp.maximum(ga[...] + gb[...] + _dot(ev, w1c[...]), 0.0)
    z = jnp.maximum(_dot(z, w2[...]) + b2[...], 0.0)
    z = _dot(z, w3[...]) + b3[...]
    out[...] = _ln(z, g[...], bn[...]) + ev


def _node_round_body(h, parts, v1a, v1b, c1, v2, c2, v3, c3, g, bn,
                     wa, ba, wb, h_out, a_out, b_out):
    hv = h[...]
    agg = parts[0] + parts[1]
    z = jnp.maximum(_dot(hv, v1a[...]) + _dot(agg, v1b[...]) + c1[...], 0.0)
    z = jnp.maximum(_dot(z, v2[...]) + c2[...], 0.0)
    z = _dot(z, v3[...]) + c3[...]
    hn = _ln(z, g[...], bn[...]) + hv
    h_out[...] = hn
    a_out[...] = _dot(hn, wa[...]) + ba[...]
    b_out[...] = _dot(hn, wb[...])


def _node_last_body(h, parts, v1a, v1b, c1, v2, c2, v3, c3, g, bn, h_out):
    hv = h[...]
    agg = parts[0] + parts[1]
    z = jnp.maximum(_dot(hv, v1a[...]) + _dot(agg, v1b[...]) + c1[...], 0.0)
    z = jnp.maximum(_dot(z, v2[...]) + c2[...], 0.0)
    z = _dot(z, v3[...]) + c3[...]
    h_out[...] = _ln(z, g[...], bn[...]) + hv


def _decoder_body(h, w1, b1, w2, b2, w3, b3, out):
    z = jnp.maximum(_dot(h[...], w1[...]) + b1[...], 0.0)
    z = jnp.maximum(_dot(z, w2[...]) + b2[...], 0.0)
    out[...] = _dot(z, w3[...]) + b3[...]


def _wspecs(n):
    return [_full_spec((HH, HH)) if s == "w" else _full_spec((1, HH)) for s in n]


def _call_edge_enc(ea, p):
    grid = (EE // EBLK,)
    return pl.pallas_call(
        _edge_enc_body,
        grid=grid,
        in_specs=[_row_spec(EBLK, 16), _full_spec((16, HH))] + _wspecs("bwbwbbb"),
        out_specs=_row_spec(EBLK, HH),
        out_shape=jax.ShapeDtypeStruct((EE, HH), _F32),
    )(ea, *p)


def _call_node_enc(x, p):
    grid = (NN // NBLK,)
    spec = _row_spec(NBLK, HH)
    return pl.pallas_call(
        _node_enc_body,
        grid=grid,
        in_specs=[spec] + _wspecs("wbwbwbbb") + _wspecs("wbw"),
        out_specs=[spec, spec, spec],
        out_shape=[jax.ShapeDtypeStruct((NN, HH), _F32)] * 3,
    )(x, *p)


def _call_edge_round(ga, gb, e, p):
    grid = (EE // EBLK,)
    spec = _row_spec(EBLK, HH)
    return pl.pallas_call(
        _edge_round_body,
        grid=grid,
        in_specs=[spec, spec, spec] + _wspecs("wwbwbbb"),
        out_specs=spec,
        out_shape=jax.ShapeDtypeStruct((EE, HH), _F32),
    )(ga, gb, e, *p)


def _call_node_round(h, parts, p, last):
    grid = (NN // NBLK,)
    spec = _row_spec(NBLK, HH)
    pspec = pl.BlockSpec((2, NBLK, HH), lambda i: (0, i, 0))
    if last:
        return pl.pallas_call(
            _node_last_body,
            grid=grid,
            in_specs=[spec, pspec] + _wspecs("wwbwbwbbb"),
            out_specs=spec,
            out_shape=jax.ShapeDtypeStruct((NN, HH), _F32),
        )(h, parts, *p)
    return pl.pallas_call(
        _node_round_body,
        grid=grid,
        in_specs=[spec, pspec] + _wspecs("wwbwbwbbb") + _wspecs("wbw"),
        out_specs=[spec, spec, spec],
        out_shape=[jax.ShapeDtypeStruct((NN, HH), _F32)] * 3,
    )(h, parts, *p)


def _call_decoder(h, p):
    grid = (NN // NBLK,)
    spec = _row_spec(NBLK, HH)
    return pl.pallas_call(
        _decoder_body,
        grid=grid,
        in_specs=[spec] + _wspecs("wbwbwb"),
        out_specs=spec,
        out_shape=jax.ShapeDtypeStruct((NN, HH), _F32),
    )(h, *p)


def _sc_mesh():
    return plsc.VectorSubcoreMesh(
        core_axis_name="c", subcore_axis_name="s", num_cores=NC, num_subcores=NS)


NCH = PER_W // CH
NBUF = 4
NBUF_S = 3


def _sc_gather_body(hs_a, hs_b, src3, dst3, ga_out, gb_out,
                    idxs, idxd, bufa, bufb, ga_sem, gb_sem, wa_sem, wb_sem):
    wid = lax.axis_index("s") * NC + lax.axis_index("c")
    base = wid * PER_W

    pltpu.sync_copy(src3.at[wid], idxs)
    pltpu.sync_copy(dst3.at[wid], idxd)

    def issue_g(j, p):
        pltpu.async_copy(hs_a.at[idxs.at[j]], bufa.at[p], ga_sem.at[p])
        pltpu.async_copy(hs_b.at[idxd.at[j]], bufb.at[p], gb_sem.at[p])

    def wait_g(p):
        pltpu.make_async_copy(hs_a.at[pl.ds(0, CH)], bufa.at[p], ga_sem.at[p]).wait()
        pltpu.make_async_copy(hs_b.at[pl.ds(0, CH)], bufb.at[p], gb_sem.at[p]).wait()

    def issue_w(j, p):
        off = base + j * CH
        pltpu.async_copy(bufa.at[p], ga_out.at[pl.ds(off, CH)], wa_sem.at[p])
        pltpu.async_copy(bufb.at[p], gb_out.at[pl.ds(off, CH)], wb_sem.at[p])

    def wait_w(p):
        pltpu.make_async_copy(bufa.at[p], ga_out.at[pl.ds(0, CH)], wa_sem.at[p]).wait()
        pltpu.make_async_copy(bufb.at[p], gb_out.at[pl.ds(0, CH)], wb_sem.at[p]).wait()

    issue_g(0, 0)

    def body(i, carry):
        j = i + 1
        p = lax.rem(j, NBUF)
        q = lax.rem(j - 1, NBUF)

        @pl.when(j >= NBUF)
        def _():
            wait_w(p)

        issue_g(j, p)
        wait_g(q)
        issue_w(j - 1, q)
        return carry

    lax.fori_loop(0, NCH - 1, body, 0)

    q = (NCH - 1) % NBUF
    wait_g(q)
    issue_w(NCH - 1, q)
    for p in range(NBUF):
        wait_w(p)


def _sc_gather(hs_a, hs_b, src3, dst3):
    k = pl.kernel(
        _sc_gather_body,
        out_type=[
            jax.ShapeDtypeStruct((EE, HH), _F32),
            jax.ShapeDtypeStruct((EE, HH), _F32),
        ],
        mesh=_sc_mesh(),
        scratch_types=[
            pltpu.VMEM((NCH, CH), jnp.int32),
            pltpu.VMEM((NCH, CH), jnp.int32),
            pltpu.VMEM((NBUF, CH, HH), _F32),
            pltpu.VMEM((NBUF, CH, HH), _F32),
            pltpu.SemaphoreType.DMA((NBUF,)),
            pltpu.SemaphoreType.DMA((NBUF,)),
            pltpu.SemaphoreType.DMA((NBUF,)),
            pltpu.SemaphoreType.DMA((NBUF,)),
        ],
    )
    return k(hs_a, hs_b, src3, dst3)


def _sc_scatter_body(e_new, dst3, zeros, out, idxd, rows, agg_sh,
                     ld_sem, sc_sem):
    cid = lax.axis_index("c")
    sid = lax.axis_index("s")
    wid = sid * NC + cid
    base = wid * PER_W

    pltpu.sync_copy(zeros, agg_sh.at[pl.ds(sid * N_PER_S, N_PER_S)])
    pltpu.sync_copy(dst3.at[wid], idxd)
    plsc.subcore_barrier()

    def issue_ld(j, p):
        off = base + j * CH
        pltpu.async_copy(e_new.at[pl.ds(off, CH)], rows.at[p], ld_sem.at[p])

    def wait_ld(p):
        pltpu.make_async_copy(e_new.at[pl.ds(0, CH)], rows.at[p], ld_sem.at[p]).wait()

    def issue_sc(j, p):
        pltpu.async_copy(rows.at[p], agg_sh.at[idxd.at[j]], sc_sem.at[p], add=True)

    def wait_sc(p):
        pltpu.make_async_copy(rows.at[p], agg_sh.at[pl.ds(0, CH)], sc_sem.at[p]).wait()

    issue_ld(0, 0)

    def body(i, carry):
        j = i + 1
        p = lax.rem(j, NBUF_S)
        q = lax.rem(j - 1, NBUF_S)

        @pl.when(j >= NBUF_S)
        def _():
            wait_sc(p)

        issue_ld(j, p)
        wait_ld(q)
        issue_sc(j - 1, q)
        return carry

    lax.fori_loop(0, NCH - 1, body, 0)

    q = (NCH - 1) % NBUF_S
    wait_ld(q)
    issue_sc(NCH - 1, q)
    for p in range(NBUF_S):
        wait_sc(p)

    plsc.subcore_barrier()
    pltpu.sync_copy(
        agg_sh.at[pl.ds(sid * N_PER_S, N_PER_S)],
        out.at[cid, pl.ds(sid * N_PER_S, N_PER_S)],
    )


def _sc_scatter(e_new, dst3, zeros):
    k = pl.kernel(
        _sc_scatter_body,
        out_type=jax.ShapeDtypeStruct((NC, NPAD, HH), _F32),
        mesh=_sc_mesh(),
        scratch_types=[
            pltpu.VMEM((NCH, CH), jnp.int32),
            pltpu.VMEM((NBUF_S, CH, HH), _F32),
            pltpu.VMEM_SHARED((NPAD, HH), _F32),
            pltpu.SemaphoreType.DMA((NBUF_S,)),
            pltpu.SemaphoreType.DMA((NBUF_S,)),
        ],
    )
    return k(e_new, dst3, zeros)


def _mlp_params(p, ln):
    ls = p["layers"]
    out = []
    for l in ls:
        out.append(l["W"])
        out.append(l["b"].reshape(1, -1))
    if ln:
        out.append(p["ln"]["g"].reshape(1, -1))
        out.append(p["ln"]["b"].reshape(1, -1))
    return out


def kernel(x, edge_index, edge_attr, params):
    src = edge_index[0]
    dst = edge_index[1]

    enc_e = _mlp_params(params["edge_enc"], True)
    enc_n = _mlp_params(params["node_enc"], True)
    dec = _mlp_params(params["decoder"], False)
    w3d = jnp.zeros((HH, HH), _F32).at[:, :3].set(dec[4])
    b3d = jnp.zeros((1, HH), _F32).at[:, :3].set(dec[5])
    dec = dec[:4] + [w3d, b3d]

    blocks = []
    for bp in params["blocks"]:
        em = _mlp_params(bp["edge_mlp"], True)
        w1 = em[0]
        blk = {
            "wa": w1[:HH],
            "ba": em[1],
            "wb": w1[HH:2 * HH],
            "edge": [w1[2 * HH:]] + em[2:],
        }
        nm = _mlp_params(bp["node_mlp"], True)
        v1 = nm[0]
        blk["node"] = [v1[:HH], v1[HH:]] + nm[1:]
        blocks.append(blk)

    e = _call_edge_enc(edge_attr, enc_e)
    b0 = blocks[0]
    h, hs_a, hs_b = _call_node_enc(x, enc_n + [b0["wa"], b0["ba"], b0["wb"]])

    zeros = jnp.zeros((N_PER_S, HH), _F32)
    src3 = src.reshape(NW, NCH, CH)
    dst3 = dst.reshape(NW, NCH, CH)

    for r in range(15):
        blk = blocks[r]
        ga, gb = _sc_gather(hs_a, hs_b, src3, dst3)
        e = _call_edge_round(ga, gb, e, blk["edge"])
        parts = _sc_scatter(e, dst3, zeros)[:, :NN]
        if r + 1 < 15:
            nxt = blocks[r + 1]
            h, hs_a, hs_b = _call_node_round(
                h, parts, blk["node"] + [nxt["wa"], nxt["ba"], nxt["wb"]], False)
        else:
            h = _call_node_round(h, parts, blk["node"], True)

    out = _call_decoder(h, dec)
    return out[:, :3]
